# Initial kernel scaffold; baseline (speedup 1.0000x reference)
import jax
import jax.numpy as jnp
from jax.experimental import pallas as pl


def kernel(x, edge_index, W1, b1):
    raise NotImplementedError("write your pallas kernel here")


# speedup vs baseline: 5.5279x; 5.5279x over previous
import functools

import jax
import jax.numpy as jnp
from jax import lax
from jax.experimental import pallas as pl
from jax.experimental.pallas import tpu as pltpu
from jax.experimental.pallas import tpu_sc as plsc

_C = 1.0
_MIN_NORM = 1e-15
_NC = 2
_NS = 16
_CH = 128


def _norm(v):
    return jnp.maximum(jnp.sqrt(jnp.sum(v * v, axis=-1, keepdims=True)), _MIN_NORM)


def _artanh(v):
    v = jnp.clip(v, -1.0 + 1e-7, 1.0 - 1e-7)
    return 0.5 * jnp.log((1.0 + v) / (1.0 - v))


def _proj(v):
    n = _norm(v)
    maxnorm = 1.0 - 1e-5
    return jnp.where(n > maxnorm, v / n * maxnorm, v)


def _expmap0(u):
    un = _norm(u)
    return jnp.tanh(un) * u / un


def _logmap0(p):
    pn = _norm(p)
    return _artanh(pn) * p / pn


def _pre_body(x_ref, wt_ref, b_ref, o_ref):
    x = _proj(x_ref[...])
    xn = _norm(x)
    mx = jnp.dot(x, wt_ref[...], preferred_element_type=jnp.float32)
    mxn = _norm(mx)
    h = jnp.tanh(mxn / xn * _artanh(xn)) * mx / mxn
    h = _proj(h)
    hb = _proj(_expmap0(b_ref[...]))
    x2 = jnp.sum(h * h, axis=-1, keepdims=True)
    y2 = jnp.sum(hb * hb, axis=-1, keepdims=True)
    xy = jnp.sum(h * hb, axis=-1, keepdims=True)
    num = (1.0 + 2.0 * xy + y2) * h + (1.0 - x2) * hb
    den = 1.0 + 2.0 * xy + x2 * y2
    h = _proj(num / jnp.maximum(den, _MIN_NORM))
    o_ref[...] = _logmap0(h)


def _post_body(s0_ref, s1_ref, deg_ref, o_ref):
    s = s0_ref[...] + s1_ref[...]
    deg = jnp.maximum(jnp.sum(deg_ref[...], axis=-1, keepdims=True), 1.0)
    agg = s / deg
    h = _proj(_expmap0(agg))
    ht = jax.nn.relu(_logmap0(h))
    o_ref[...] = _proj(_expmap0(ht))


def _make_sc_agg(N, N_pad, D, K):
    rows_per_tile = N_pad // _NS
    mesh = plsc.VectorSubcoreMesh(core_axis_name="c", subcore_axis_name="s")

    @functools.partial(
        pl.kernel,
        out_type=(
            jax.ShapeDtypeStruct((_NC, N_pad, D), jnp.float32),
            jax.ShapeDtypeStruct((_NC * _NS, N_pad), jnp.float32),
        ),
        mesh=mesh,
        compiler_params=pltpu.CompilerParams(needs_layout_passes=False),
        scratch_types=[
            pltpu.VMEM((K, _CH), jnp.int32),
            pltpu.VMEM((K, _CH), jnp.int32),
            pltpu.VMEM((_CH, D), jnp.float32),
            pltpu.VMEM((N_pad,), jnp.float32),
            pltpu.VMEM_SHARED((N_pad, D), jnp.float32),
        ],
    )
    def sc_agg(xt_hbm, src_hbm, dst_hbm, sums_hbm, deg_hbm,
               src_v, dst_v, rows_v, hist_v, acc_sh):
        cid = lax.axis_index("c")
        sid = lax.axis_index("s")
        wid = cid * _NS + sid

        pltpu.sync_copy(src_hbm.at[wid], src_v)
        pltpu.sync_copy(dst_hbm.at[wid], dst_v)

        z16 = jnp.zeros((16,), jnp.float32)

        def _zero_rows(i, _):
            for j in range(D // 16):
                rows_v[i, pl.ds(j * 16, 16)] = z16
            return ()
        lax.fori_loop(0, _CH, _zero_rows, ())
        base = sid * rows_per_tile
        off = 0
        while off < rows_per_tile:
            sz = min(_CH, rows_per_tile - off)
            pltpu.sync_copy(rows_v.at[pl.ds(0, sz)],
                            acc_sh.at[pl.ds(base + off, sz)])
            off += sz

        def _zero_hist(i, _):
            hist_v[pl.ds(i * 16, 16)] = jnp.zeros((16,), jnp.float32)
            return ()
        lax.fori_loop(0, N_pad // 16, _zero_hist, ())

        plsc.subcore_barrier()

        ones16 = jnp.ones((16,), jnp.float32)

        def _chunk(k, _):
            pltpu.sync_copy(xt_hbm.at[src_v.at[k]], rows_v)
            pltpu.sync_copy(rows_v, acc_sh.at[dst_v.at[k]], add=True)
            for j in range(_CH // 16):
                idx16 = dst_v[k, pl.ds(j * 16, 16)]
                plsc.addupdate_scatter(hist_v, [idx16], ones16)
            return ()

        lax.fori_loop(0, K, _chunk, ())

        plsc.subcore_barrier()

        off = 0
        while off < rows_per_tile:
            sz = min(_CH, rows_per_tile - off)
            pltpu.sync_copy(acc_sh.at[pl.ds(base + off, sz)],
                            rows_v.at[pl.ds(0, sz)])
            pltpu.sync_copy(rows_v.at[pl.ds(0, sz)],
                            sums_hbm.at[cid].at[pl.ds(base + off, sz)])
            off += sz

        pltpu.sync_copy(hist_v, deg_hbm.at[wid])

    return sc_agg


def _ceil_to(a, m):
    return (a + m - 1) // m * m


def kernel(x, edge_index, W1, b1):
    N, D = x.shape
    E = edge_index.shape[1]
    NW = _NC * _NS

    BN = 2000 if N % 2000 == 0 else 8
    grid = (N // BN,)
    xt = pl.pallas_call(
        _pre_body,
        grid=grid,
        in_specs=[
            pl.BlockSpec((BN, D), lambda i: (i, 0)),
            pl.BlockSpec((D, D), lambda i: (0, 0)),
            pl.BlockSpec((1, D), lambda i: (0, 0)),
        ],
        out_specs=pl.BlockSpec((BN, D), lambda i: (i, 0)),
        out_shape=jax.ShapeDtypeStruct((N, D), jnp.float32),
    )(x, W1.T, b1.reshape(1, D))

    K = _ceil_to(E, NW * _CH) // (NW * _CH)
    E_pad = NW * K * _CH
    rows_per_tile = _ceil_to(N + 1, _NS * 8) // _NS
    N_pad = rows_per_tile * _NS

    src = edge_index[0]
    dst = edge_index[1]
    src_r = jnp.pad(src, (0, E_pad - E)).reshape(NW, K, _CH)
    dst_r = jnp.pad(dst, (0, E_pad - E), constant_values=N).reshape(NW, K, _CH)

    sums, degs = _make_sc_agg(N, N_pad, D, K)(xt, src_r, dst_r)

    s0 = sums[0, :N]
    s1 = sums[1, :N]
    deg_t = degs.T[:N]
    out = pl.pallas_call(
        _post_body,
        grid=grid,
        in_specs=[
            pl.BlockSpec((BN, D), lambda i: (i, 0)),
            pl.BlockSpec((BN, D), lambda i: (i, 0)),
            pl.BlockSpec((BN, NW), lambda i: (i, 0)),
            ],
        out_specs=pl.BlockSpec((BN, D), lambda i: (i, 0)),
        out_shape=jax.ShapeDtypeStruct((N, D), jnp.float32),
    )(s0, s1, deg_t)

    return out, edge_index
